# baseline (device time: 93770 ns/iter reference)
import jax
import jax.numpy as jnp
from jax import lax
from jax.experimental import pallas as pl
from jax.experimental.pallas import tpu as pltpu

SEQ = 1024
H = 16
D = 128
HD = H * D
SCALE = D ** -0.5
LOG2E = 1.4426950408889634

CH = 8
CROWS = (SEQ // 2) // CH
NMSG = 2 * CH
MESH_ID = pl.DeviceIdType.MESH


def _attn_body(q_ref, k_ref, v_ref, out_ref, k_rem, v_rem, den_ref,
               p1_send, p1_recv, p2_send, p2_recv):
    my_x = lax.axis_index("x")
    my_y = lax.axis_index("y")
    nbr_x = (1 - my_x, my_y)
    nbr_y = (my_x, 1 - my_y)

    barrier_sem = pltpu.get_barrier_semaphore()
    pl.semaphore_signal(barrier_sem, inc=1, device_id=nbr_x, device_id_type=MESH_ID)
    pl.semaphore_signal(barrier_sem, inc=1, device_id=nbr_y, device_id_type=MESH_ID)
    pl.semaphore_wait(barrier_sem, 2)

    half = my_y * (SEQ // 2)
    tensors = ((k_ref, k_rem), (v_ref, v_rem))

    def p1_msg(m):
        t, c = m % 2, m // 2
        src_ref, rem_ref = tensors[t]
        rows = pl.ds(half + c * CROWS, CROWS)
        return pltpu.make_async_remote_copy(
            src_ref=src_ref.at[rows],
            dst_ref=rem_ref.at[rows],
            send_sem=p1_send.at[t, c],
            recv_sem=p1_recv.at[t, c],
            device_id=nbr_x,
            device_id_type=MESH_ID,
        )

    def p2_msg(m):
        t, c = m % 2, m // 2
        rem_ref = tensors[t][1]
        rows = pl.ds(half + c * CROWS, CROWS)
        return pltpu.make_async_remote_copy(
            src_ref=rem_ref.at[rows],
            dst_ref=rem_ref.at[rows],
            send_sem=p2_send.at[t, c],
            recv_sem=p2_recv.at[t, c],
            device_id=nbr_y,
            device_id_type=MESH_ID,
        )

    msgs1 = [p1_msg(m) for m in range(NMSG)]
    msgs2 = [p2_msg(m) for m in range(NMSG)]
    for msg in msgs1:
        msg.start()

    ones = jnp.ones((SEQ, 1), jnp.bfloat16)

    out_ref[...] = jnp.zeros((SEQ, HD), jnp.float32)
    den_ref[...] = jnp.zeros((H, SEQ, 1), jnp.float32)

    def a_head(h):
        sl = slice(h * D, (h + 1) * D)
        s1 = lax.dot_general(
            q_ref[:, sl], k_ref[:, sl], (((1,), (1,)), ((), ())),
            preferred_element_type=jnp.float32,
        )
        e1 = jnp.exp2(s1).astype(jnp.bfloat16)
        den_ref[h] = den_ref[h] + lax.dot_general(
            e1, ones, (((1,), (0,)), ((), ())),
            preferred_element_type=jnp.float32,
        )
        out_ref[:, sl] = out_ref[:, sl] + lax.dot_general(
            e1, v_ref[:, sl], (((1,), (0,)), ((), ())),
            preferred_element_type=jnp.float32,
        )

    QROWS = SEQ // 4
    ones_q = jnp.ones((QROWS, 1), jnp.bfloat16)

    def b_slab(h, rows, normalize=False):
        sl = slice(h * D, (h + 1) * D)
        s2 = lax.dot_general(
            q_ref[:, sl], k_rem[rows, sl], (((1,), (1,)), ((), ())),
            preferred_element_type=jnp.float32,
        )
        e2 = jnp.exp2(s2).astype(jnp.bfloat16)
        den = den_ref[h] + lax.dot_general(
            e2, ones_q, (((1,), (0,)), ((), ())),
            preferred_element_type=jnp.float32,
        )
        acc = out_ref[:, sl] + lax.dot_general(
            e2, v_rem[rows, sl], (((1,), (0,)), ((), ())),
            preferred_element_type=jnp.float32,
        )
        if normalize:
            out_ref[:, sl] = acc / den
        else:
            den_ref[h] = den
            out_ref[:, sl] = acc

    def wait_fwd(m):
        msgs1[m].wait_recv()
        msgs2[m].start()

    other = (SEQ // 2) - half
    for h in range(8):
        a_head(h)
        wait_fwd(h)
    for h in range(H):
        b_slab(h, pl.ds(half, QROWS))
    for m in range(8, 11):
        wait_fwd(m)
    for h in range(8, 12):
        a_head(h)
    wait_fwd(11)
    wait_fwd(12)
    for m in range(8):
        msgs2[m].wait_recv()
    for h in range(H):
        b_slab(h, pl.ds(other, QROWS))
    wait_fwd(13)
    wait_fwd(14)
    for h in range(12, H):
        a_head(h)
    wait_fwd(15)
    for h in range(H):
        b_slab(h, pl.ds(half + QROWS, QROWS))
    for m in range(8, NMSG):
        msgs2[m].wait_recv()
    for h in range(H):
        b_slab(h, pl.ds(other + QROWS, QROWS), normalize=True)

    for msg in msgs1:
        msg.wait_send()
    for msg in msgs2:
        msg.wait_send()


def kernel(Q, K, V):
    q = (jnp.reshape(Q, (SEQ, HD)) * (SCALE * LOG2E)).astype(jnp.bfloat16)
    k = jnp.reshape(K, (SEQ, HD)).astype(jnp.bfloat16)
    v = jnp.reshape(V, (SEQ, HD)).astype(jnp.bfloat16)

    out = pl.pallas_call(
        _attn_body,
        out_shape=jax.ShapeDtypeStruct((SEQ, HD), jnp.float32),
        in_specs=[
            pl.BlockSpec(memory_space=pltpu.VMEM),
            pl.BlockSpec(memory_space=pltpu.VMEM),
            pl.BlockSpec(memory_space=pltpu.VMEM),
        ],
        out_specs=pl.BlockSpec(memory_space=pltpu.VMEM),
        scratch_shapes=[
            pltpu.VMEM((SEQ, HD), jnp.bfloat16),
            pltpu.VMEM((SEQ, HD), jnp.bfloat16),
            pltpu.VMEM((H, SEQ, 1), jnp.float32),
            pltpu.SemaphoreType.DMA((2, CH)),
            pltpu.SemaphoreType.DMA((2, CH)),
            pltpu.SemaphoreType.DMA((2, CH)),
            pltpu.SemaphoreType.DMA((2, CH)),
        ],
        compiler_params=pltpu.CompilerParams(collective_id=0),
    )(q, k, v)
    return jnp.reshape(out, (1, SEQ, H, D))
